# Initial kernel scaffold; baseline (speedup 1.0000x reference)
import jax
import jax.numpy as jnp
from jax.experimental import pallas as pl


def kernel(z, embeddings, W1, b1, W2, b2, W3, b3, W4, b4):
    raise NotImplementedError("write your pallas kernel here")


# speedup vs baseline: 2.3371x; 2.3371x over previous
import jax
import jax.numpy as jnp
from jax.experimental import pallas as pl

VOCAB = 1024
E = 256
K = 8
T = 4
H = 4 * E
VB = 256
EPS = 1e-5


def _mlp_kernel(x_ref, w1_ref, b1_ref, w2_ref, b2_ref, w3_ref, b3_ref,
                w4_ref, b4_ref, out_ref):
    x = x_ref[0]
    h = jnp.maximum(
        jnp.dot(x, w1_ref[...], preferred_element_type=jnp.float32)
        + b1_ref[...], 0.0)
    h = jnp.maximum(
        jnp.dot(h, w2_ref[...], preferred_element_type=jnp.float32)
        + b2_ref[...], 0.0)
    h = jnp.maximum(
        jnp.dot(h, w3_ref[...], preferred_element_type=jnp.float32)
        + b3_ref[...], 0.0)
    m = (jnp.dot(h, w4_ref[...], preferred_element_type=jnp.float32)
         + b4_ref[...])
    mu = jnp.mean(m, axis=-1, keepdims=True)
    var = jnp.mean((m - mu) ** 2, axis=-1, keepdims=True)
    out_ref[0] = (m - mu) * jax.lax.rsqrt(var + EPS)


def _attn_kernel(z_ref, mem_ref, tok_ref, zq_ref):
    q = z_ref[0]
    mu = jnp.mean(q, axis=-1, keepdims=True)
    var = jnp.mean((q - mu) ** 2, axis=-1, keepdims=True)
    qn = (q - mu) * jax.lax.rsqrt(var + EPS) * (E ** -0.5)
    mem = mem_ref[0]
    s = jax.lax.dot_general(qn, mem, (((1,), (1,)), ((), ())),
                            preferred_element_type=jnp.float32)
    mx = jnp.max(s, axis=-1, keepdims=True)
    e = jnp.exp(s - mx)
    p = e / jnp.sum(e, axis=-1, keepdims=True)
    pm = jnp.max(p, axis=-1, keepdims=True)
    idx = jax.lax.broadcasted_iota(jnp.int32, s.shape, 1)
    tok = jnp.min(jnp.where(p == pm, idx, VOCAB), axis=-1)
    tok_ref[0, 0, :] = tok
    zq_ref[0] = jax.lax.dot_general(p, mem, (((1,), (0,)), ((), ())),
                                    preferred_element_type=jnp.float32)


@jax.jit
def kernel(z, embeddings, W1, b1, W2, b2, W3, b3, W4, b4):
    bk = z.shape[0] // T

    mem = pl.pallas_call(
        _mlp_kernel,
        grid=(T, VOCAB // VB),
        in_specs=[
            pl.BlockSpec((1, VB, E), lambda t, v: (t, v, 0)),
            pl.BlockSpec((E, H), lambda t, v: (0, 0)),
            pl.BlockSpec((1, H), lambda t, v: (0, 0)),
            pl.BlockSpec((H, H), lambda t, v: (0, 0)),
            pl.BlockSpec((1, H), lambda t, v: (0, 0)),
            pl.BlockSpec((H, H), lambda t, v: (0, 0)),
            pl.BlockSpec((1, H), lambda t, v: (0, 0)),
            pl.BlockSpec((H, E), lambda t, v: (0, 0)),
            pl.BlockSpec((1, E), lambda t, v: (0, 0)),
        ],
        out_specs=pl.BlockSpec((1, VB, E), lambda t, v: (t, v, 0)),
        out_shape=jax.ShapeDtypeStruct((T, VOCAB, E), jnp.float32),

    )(embeddings.reshape(VOCAB, T, E).transpose(1, 0, 2),
      W1, b1.reshape(1, H), W2, b2.reshape(1, H),
      W3, b3.reshape(1, H), W4, b4.reshape(1, E))

    tok, zq = pl.pallas_call(
        _attn_kernel,
        grid=(T,),
        in_specs=[
            pl.BlockSpec((1, bk, E), lambda t: (t, 0, 0)),
            pl.BlockSpec((1, VOCAB, E), lambda t: (t, 0, 0)),
        ],
        out_specs=[
            pl.BlockSpec((1, 1, bk), lambda t: (t, 0, 0)),
            pl.BlockSpec((1, bk, E), lambda t: (t, 0, 0)),
        ],
        out_shape=[
            jax.ShapeDtypeStruct((T, 1, bk), jnp.int32),
            jax.ShapeDtypeStruct((T, bk, E), jnp.float32),
        ],

    )(z.reshape(bk, T, E).transpose(1, 0, 2), mem)

    tokens = tok.reshape(T, bk).T.reshape(bk * T)
    z_q = zq.transpose(1, 0, 2).reshape(bk * T, E)
    return (tokens, z_q)


# speedup vs baseline: 2.7847x; 1.1915x over previous
import jax
import jax.numpy as jnp
from jax.experimental import pallas as pl
from jax.experimental.pallas import tpu as pltpu

VOCAB = 1024
E = 256
K = 8
T = 4
H = 4 * E
VB = 256
NV = VOCAB // VB
EPS = 1e-5


def _layernorm(x):
    mu = jnp.mean(x, axis=-1, keepdims=True)
    var = jnp.mean((x - mu) ** 2, axis=-1, keepdims=True)
    return (x - mu) * jax.lax.rsqrt(var + EPS)


def _fused_kernel(emb_ref, z_ref, w1_ref, b1_ref, w2_ref, b2_ref,
                  w3_ref, b3_ref, w4_ref, b4_ref, tok_ref, zq_ref, mem_s):
    v = pl.program_id(0)
    x = jnp.concatenate(
        [emb_ref[:, t * E:(t + 1) * E] for t in range(T)], axis=0)
    h = jnp.maximum(
        jnp.dot(x, w1_ref[...], preferred_element_type=jnp.float32)
        + b1_ref[...], 0.0)
    h = jnp.maximum(
        jnp.dot(h, w2_ref[...], preferred_element_type=jnp.float32)
        + b2_ref[...], 0.0)
    h = jnp.maximum(
        jnp.dot(h, w3_ref[...], preferred_element_type=jnp.float32)
        + b3_ref[...], 0.0)
    m = (jnp.dot(h, w4_ref[...], preferred_element_type=jnp.float32)
         + b4_ref[...])
    m = _layernorm(m)
    for t in range(T):
        mem_s[t, pl.ds(v * VB, VB), :] = m[t * VB:(t + 1) * VB, :]

    @pl.when(v == NV - 1)
    def _attention():
        toks = []
        for t in range(T):
            q = z_ref[:, t * E:(t + 1) * E]
            qn = _layernorm(q) * (E ** -0.5)
            memt = mem_s[t]
            s = jax.lax.dot_general(
                qn, memt, (((1,), (1,)), ((), ())),
                preferred_element_type=jnp.float32)
            mx = jnp.max(s, axis=-1, keepdims=True)
            e = jnp.exp(s - mx)
            rcp = 1.0 / jnp.sum(e, axis=-1, keepdims=True)
            em = jnp.max(e, axis=-1, keepdims=True)
            idx = jax.lax.broadcasted_iota(jnp.int32, s.shape, 1)
            toks.append(jnp.min(jnp.where(e == em, idx, VOCAB),
                                axis=-1, keepdims=True))
            o = jax.lax.dot_general(
                e, memt, (((1,), (0,)), ((), ())),
                preferred_element_type=jnp.float32) * rcp
            zq_ref[:, t, :] = o
        tok_ref[...] = jnp.concatenate(toks, axis=1)


@jax.jit
def kernel(z, embeddings, W1, b1, W2, b2, W3, b3, W4, b4):
    bk = z.shape[0] // T

    tok, zq = pl.pallas_call(
        _fused_kernel,
        grid=(NV,),
        in_specs=[
            pl.BlockSpec((VB, T * E), lambda v: (v, 0)),
            pl.BlockSpec((bk, T * E), lambda v: (0, 0)),
            pl.BlockSpec((E, H), lambda v: (0, 0)),
            pl.BlockSpec((1, H), lambda v: (0, 0)),
            pl.BlockSpec((H, H), lambda v: (0, 0)),
            pl.BlockSpec((1, H), lambda v: (0, 0)),
            pl.BlockSpec((H, H), lambda v: (0, 0)),
            pl.BlockSpec((1, H), lambda v: (0, 0)),
            pl.BlockSpec((H, E), lambda v: (0, 0)),
            pl.BlockSpec((1, E), lambda v: (0, 0)),
        ],
        out_specs=[
            pl.BlockSpec((bk, T), lambda v: (0, 0)),
            pl.BlockSpec((bk, T, E), lambda v: (0, 0, 0)),
        ],
        out_shape=[
            jax.ShapeDtypeStruct((bk, T), jnp.int32),
            jax.ShapeDtypeStruct((bk, T, E), jnp.float32),
        ],
        scratch_shapes=[pltpu.VMEM((T, VOCAB, E), jnp.float32)],
    )(embeddings.reshape(VOCAB, T * E), z.reshape(bk, T * E),
      W1, b1.reshape(1, H), W2, b2.reshape(1, H),
      W3, b3.reshape(1, H), W4, b4.reshape(1, E))

    return (tok.reshape(bk * T), zq.reshape(bk * T, E))
